# Initial kernel scaffold; baseline (speedup 1.0000x reference)
import jax
import jax.numpy as jnp
from jax.experimental import pallas as pl


def kernel(inputs, W_shared, b_shared, W_cls, b_cls, W_reg, b_reg):
    raise NotImplementedError("write your pallas kernel here")


# speedup vs baseline: 1.2525x; 1.2525x over previous
import jax
import jax.numpy as jnp
from jax.experimental import pallas as pl
from jax.experimental.pallas import tpu as pltpu

_H = 64
_W = 64
_WP = _W + 2
_ROWS = _H * _WP
_XROWS = (_H + 3) * _WP
_CIN = 256
_CMID = 512


def _rpn_body(x_ref, w9_ref, bs_ref, wb_ref, bb_ref, lg_ref, pb_ref, dl_ref):
    x = x_ref[0]
    acc = jnp.zeros((_ROWS, _CMID), jnp.float32)
    for k in range(9):
        off = (k // 3) * _WP + (k % 3)
        xs = jax.lax.slice(x, (off, 0), (off + _ROWS, _CIN))
        acc = acc + jnp.dot(xs, w9_ref[k], preferred_element_type=jnp.float32)
    shared = jnp.maximum(acc + bs_ref[0], 0.0).astype(jnp.bfloat16)
    y = jnp.dot(shared, wb_ref[...], preferred_element_type=jnp.float32) + bb_ref[0]
    probs = jax.nn.sigmoid(y[:, 18:24])
    for h in range(_H):
        base = h * _WP
        seg = jax.lax.slice(y, (base, 0), (base + _W, 18))
        lg_ref[0, pl.ds(h * _W, _W), :] = seg[:, 0:6]
        dl_ref[0, pl.ds(h * _W, _W), :] = seg[:, 6:18]
        pb_ref[0, pl.ds(h * _W, _W), :] = jax.lax.slice(
            probs, (base, 0), (base + _W, 6))


def kernel(inputs, W_shared, b_shared, W_cls, b_cls, W_reg, b_reg):
    B = inputs.shape[0]
    x_pad = jnp.pad(inputs, ((0, 0), (1, 2), (1, 1), (0, 0)))
    x_flat = x_pad.reshape(B, _XROWS, _CIN).astype(jnp.bfloat16)

    w9 = W_shared.reshape(9, _CIN, _CMID).astype(jnp.bfloat16)
    bs = b_shared.reshape(1, _CMID)

    wc = W_cls.reshape(_CMID, 6)
    wr = W_reg.reshape(_CMID, 12)
    wc3 = wc.reshape(_CMID, 3, 2)
    wdiff = wc3[:, :, 0] - wc3[:, :, 1]
    wd = jnp.stack([wdiff, -wdiff], axis=-1).reshape(_CMID, 6)
    wbig = jnp.concatenate([wc, wr, wd], axis=1).astype(jnp.bfloat16)

    bc3 = b_cls.reshape(3, 2)
    bdiff = bc3[:, 0] - bc3[:, 1]
    bd = jnp.stack([bdiff, -bdiff], axis=-1).reshape(6)
    bbig = jnp.concatenate([b_cls, b_reg, bd]).reshape(1, 24)

    n_pix = _H * _W
    grid_spec = pl.GridSpec(
        grid=(B,),
        in_specs=[
            pl.BlockSpec((1, _XROWS, _CIN), lambda b: (b, 0, 0)),
            pl.BlockSpec((9, _CIN, _CMID), lambda b: (0, 0, 0)),
            pl.BlockSpec((1, _CMID), lambda b: (0, 0)),
            pl.BlockSpec((_CMID, 24), lambda b: (0, 0)),
            pl.BlockSpec((1, 24), lambda b: (0, 0)),
        ],
        out_specs=[
            pl.BlockSpec((1, n_pix, 6), lambda b: (b, 0, 0)),
            pl.BlockSpec((1, n_pix, 6), lambda b: (b, 0, 0)),
            pl.BlockSpec((1, n_pix, 12), lambda b: (b, 0, 0)),
        ],
    )
    logits, probs, deltas = pl.pallas_call(
        _rpn_body,
        grid_spec=grid_spec,
        out_shape=[
            jax.ShapeDtypeStruct((B, n_pix, 6), jnp.float32),
            jax.ShapeDtypeStruct((B, n_pix, 6), jnp.float32),
            jax.ShapeDtypeStruct((B, n_pix, 12), jnp.float32),
        ],
        compiler_params=pltpu.CompilerParams(
            dimension_semantics=("arbitrary",),
        ),
    )(x_flat, w9, bs, wbig, bbig)

    rpn_class_logits = logits.reshape(B, n_pix * 3, 2)
    rpn_probs = probs.reshape(B, n_pix * 3, 2)
    rpn_deltas = deltas.reshape(B, n_pix * 3, 4)
    return (rpn_class_logits, rpn_probs, rpn_deltas)
